# speedup vs baseline: 1.2886x; 1.2886x over previous
import jax
import jax.numpy as jnp
from jax.experimental import pallas as pl

_KBLK = 2000


def _stage_a(x_ref, w_ref, t_ref, csum_ref):
    @pl.when(pl.program_id(0) == 0)
    def _init():
        t_ref[...] = jnp.zeros_like(t_ref)
        csum_ref[...] = jnp.zeros_like(csum_ref)

    xb = x_ref[...]
    wb = w_ref[...]
    t_ref[...] += jax.lax.dot_general(
        xb, wb, (((0,), (0,)), ((), ())), preferred_element_type=jnp.float32)
    csum_ref[...] += jnp.sum(wb, axis=0, keepdims=True)


def _stage_b(t_ref, csum_ref, wl1_ref, bl1_ref, adj_ref, adj2_ref,
             bgc1_ref, wgc2_ref, bgc2_ref, wgc3_ref, bgc3_ref,
             wl4_ref, bl4_ref, out_ref):
    s1 = jax.lax.dot_general(
        wl1_ref[...], t_ref[...], (((0,), (0,)), ((), ())),
        preferred_element_type=jnp.float32)
    s1 = s1 + bl1_ref[...] * csum_ref[...]
    h1 = jnp.maximum(
        jnp.dot(adj_ref[...], s1, preferred_element_type=jnp.float32)
        + bgc1_ref[...], 0.0)
    w34 = jnp.dot(wgc3_ref[...], wl4_ref[...], preferred_element_type=jnp.float32)
    w234 = jnp.dot(wgc2_ref[...], w34, preferred_element_type=jnp.float32)
    u = jnp.dot(h1, w234, preferred_element_type=jnp.float32)
    v = (jnp.dot(adj_ref[...], u, preferred_element_type=jnp.float32)
         + jnp.dot(bgc2_ref[...], w34, preferred_element_type=jnp.float32))
    w = (jnp.dot(adj2_ref[...], v, preferred_element_type=jnp.float32)
         + jnp.dot(bgc3_ref[...], wl4_ref[...], preferred_element_type=jnp.float32)
         + bl4_ref[...])
    m = jnp.max(w, axis=1, keepdims=True)
    lse = m + jnp.log(jnp.sum(jnp.exp(w - m), axis=1, keepdims=True))
    out_ref[...] = w - lse


def kernel(x, adj, adj2, W_l1, b_l1, W_gc1, b_gc1, W_gc2, b_gc2,
           W_gc3, b_gc3, W_l4, b_l4):
    nfeat = x.shape[0]
    nblk = nfeat // _KBLK
    t, csum = pl.pallas_call(
        _stage_a,
        grid=(nblk,),
        in_specs=[
            pl.BlockSpec((_KBLK, x.shape[1]), lambda i: (i, 0)),
            pl.BlockSpec((_KBLK, W_gc1.shape[1]), lambda i: (i, 0)),
        ],
        out_specs=[
            pl.BlockSpec((x.shape[1], W_gc1.shape[1]), lambda i: (0, 0)),
            pl.BlockSpec((1, W_gc1.shape[1]), lambda i: (0, 0)),
        ],
        out_shape=[
            jax.ShapeDtypeStruct((x.shape[1], W_gc1.shape[1]), jnp.float32),
            jax.ShapeDtypeStruct((1, W_gc1.shape[1]), jnp.float32),
        ],
    )(x, W_gc1)

    n = adj.shape[0]
    out = pl.pallas_call(
        _stage_b,
        out_shape=jax.ShapeDtypeStruct((n, W_l4.shape[1]), jnp.float32),
    )(t, csum, W_l1, b_l1.reshape(n, 1), adj, adj2,
      b_gc1.reshape(1, -1), W_gc2, b_gc2.reshape(1, -1),
      W_gc3, b_gc3.reshape(1, -1), W_l4, b_l4.reshape(1, -1))
    return out
